# baseline (speedup 1.0000x reference)
import functools

import jax
import jax.numpy as jnp
from jax import lax
from jax.experimental import pallas as pl
from jax.experimental.pallas import tpu as pltpu
from jax.experimental.pallas import tpu_sc as plsc

_NUM_CORES = 2
_NUM_SUBCORES = 16
_NW = _NUM_CORES * _NUM_SUBCORES
_CHUNK = 640


@functools.partial(jax.jit, static_argnames=("b", "d"))
def _sc_gather(flat_idx, table, b, d):
    b_per_w = b // _NW
    n_chunks = b_per_w // _CHUNK
    mesh = plsc.VectorSubcoreMesh(core_axis_name="c", subcore_axis_name="s")

    @functools.partial(
        pl.kernel,
        mesh=mesh,
        out_type=jax.ShapeDtypeStruct((b, d), jnp.float32),
        scratch_types=[
            pltpu.VMEM((_CHUNK,), jnp.int32),
            pltpu.VMEM((_CHUNK, d), jnp.float32),
            pltpu.SemaphoreType.DMA,
        ],
        compiler_params=pltpu.CompilerParams(use_tc_tiling_on_sc=False),
    )
    def k(idx_hbm, table_hbm, out_hbm, idx_v, rows_v, sem):
        wid = lax.axis_index("s") * _NUM_CORES + lax.axis_index("c")
        base = wid * b_per_w
        for g in range(n_chunks):
            off = base + g * _CHUNK
            pltpu.sync_copy(idx_hbm.at[pl.ds(off, _CHUNK)], idx_v)
            pltpu.async_copy(table_hbm.at[idx_v], rows_v, sem).wait()
            pltpu.sync_copy(rows_v, out_hbm.at[pl.ds(off, _CHUNK)])

    return k(flat_idx, table)


def kernel(inputs, table):
    n, s = inputs.shape
    d = table.shape[1]
    flat = inputs.reshape(n * s).astype(jnp.int32)
    out = _sc_gather(flat, table, n * s, d)
    return out.reshape(n, s, d)


# speedup vs baseline: 1.0095x; 1.0095x over previous
import functools

import jax
import jax.numpy as jnp
from jax import lax
from jax.experimental import pallas as pl
from jax.experimental.pallas import tpu as pltpu
from jax.experimental.pallas import tpu_sc as plsc

_NUM_CORES = 2
_NUM_SUBCORES = 16
_NW = _NUM_CORES * _NUM_SUBCORES
_CHUNK = 640


@functools.partial(jax.jit, static_argnames=("b", "d"))
def _sc_gather(flat_idx, table, b, d):
    b_per_w = b // _NW
    n_chunks = b_per_w // _CHUNK
    mesh = plsc.VectorSubcoreMesh(core_axis_name="c", subcore_axis_name="s")

    @functools.partial(
        pl.kernel,
        mesh=mesh,
        out_type=jax.ShapeDtypeStruct((b, d), jnp.float32),
        scratch_types=[
            pltpu.VMEM((b_per_w,), jnp.int32),
            pltpu.VMEM((2, _CHUNK, d), jnp.float32),
            pltpu.SemaphoreType.DMA,
            pltpu.SemaphoreType.DMA,
            pltpu.SemaphoreType.DMA,
            pltpu.SemaphoreType.DMA,
        ],
        compiler_params=pltpu.CompilerParams(use_tc_tiling_on_sc=False),
    )
    def k(idx_hbm, table_hbm, out_hbm, idx_v, rows_v, g0, g1, o0, o1):
        wid = lax.axis_index("s") * _NUM_CORES + lax.axis_index("c")
        base = wid * b_per_w
        gat_sems = (g0, g1)
        out_sems = (o0, o1)
        pltpu.sync_copy(idx_hbm.at[pl.ds(base, b_per_w)], idx_v)
        gathers = [None] * n_chunks
        writes = [None] * n_chunks
        for g in range(n_chunks):
            bb = g & 1
            if g >= 2:
                writes[g - 2].wait()
            gathers[g] = pltpu.async_copy(
                table_hbm.at[idx_v.at[pl.ds(g * _CHUNK, _CHUNK)]],
                rows_v.at[bb],
                gat_sems[bb],
            )
            if g >= 1:
                gathers[g - 1].wait()
                writes[g - 1] = pltpu.async_copy(
                    rows_v.at[1 - bb],
                    out_hbm.at[pl.ds(base + (g - 1) * _CHUNK, _CHUNK)],
                    out_sems[1 - bb],
                )
        last = n_chunks - 1
        gathers[last].wait()
        writes[last] = pltpu.async_copy(
            rows_v.at[last & 1],
            out_hbm.at[pl.ds(base + last * _CHUNK, _CHUNK)],
            out_sems[last & 1],
        )
        writes[last - 1].wait()
        writes[last].wait()

    return k(flat_idx, table)


def kernel(inputs, table):
    n, s = inputs.shape
    d = table.shape[1]
    flat = inputs.reshape(n * s).astype(jnp.int32)
    out = _sc_gather(flat, table, n * s, d)
    return out.reshape(n, s, d)


# speedup vs baseline: 1.0268x; 1.0172x over previous
import functools

import jax
import jax.numpy as jnp
from jax import lax
from jax.experimental import pallas as pl
from jax.experimental.pallas import tpu as pltpu
from jax.experimental.pallas import tpu_sc as plsc

_NUM_CORES = 2
_NUM_SUBCORES = 16
_NW = _NUM_CORES * _NUM_SUBCORES
_CHUNK = 640


@functools.partial(jax.jit, static_argnames=("b", "d"))
def _sc_gather(flat_idx, table, b, d):
    b_per_w = b // _NW
    n_chunks = b_per_w // _CHUNK
    mesh = plsc.VectorSubcoreMesh(core_axis_name="c", subcore_axis_name="s")

    @functools.partial(
        pl.kernel,
        mesh=mesh,
        out_type=jax.ShapeDtypeStruct((b, d), jnp.float32),
        scratch_types=[
            pltpu.VMEM((b_per_w,), jnp.int32),
            pltpu.VMEM((2, _CHUNK, d), jnp.float32),
            pltpu.SemaphoreType.DMA,
            pltpu.SemaphoreType.DMA,
            pltpu.SemaphoreType.DMA,
            pltpu.SemaphoreType.DMA,
        ],
        compiler_params=pltpu.CompilerParams(use_tc_tiling_on_sc=False),
    )
    def k(idx_hbm, table_hbm, out_hbm, idx_v, rows_v, g0, g1, o0, o1):
        wid = lax.axis_index("s") * _NUM_CORES + lax.axis_index("c")
        base = wid * b_per_w
        gat_sems = (g0, g1)
        out_sems = (o0, o1)
        pltpu.sync_copy(idx_hbm.at[pl.ds(base, b_per_w)], idx_v)
        gathers = [None] * n_chunks
        writes = [None] * n_chunks
        for g in range(n_chunks):
            bb = g & 1
            if g >= 2:
                writes[g - 2].wait()
            gathers[g] = pltpu.async_copy(
                table_hbm.at[idx_v.at[pl.ds(g * _CHUNK, _CHUNK)]],
                rows_v.at[bb],
                gat_sems[bb],
            )
            if g >= 1:
                gathers[g - 1].wait()
                writes[g - 1] = pltpu.async_copy(
                    rows_v.at[1 - bb],
                    out_hbm.at[pl.ds(base + (g - 1) * _CHUNK, _CHUNK)],
                    out_sems[1 - bb],
                )
        last = n_chunks - 1
        gathers[last].wait()
        writes[last] = pltpu.async_copy(
            rows_v.at[last & 1],
            out_hbm.at[pl.ds(base + last * _CHUNK, _CHUNK)],
            out_sems[last & 1],
        )
        writes[last - 1].wait()
        writes[last].wait()

    return k(flat_idx, table)


def kernel(inputs, table):
    n, s = inputs.shape
    d = table.shape[1]
    flat = inputs.T.reshape(n * s).astype(jnp.int32)
    out = _sc_gather(flat, table, n * s, d)
    return out.reshape(s, n, d).transpose(1, 0, 2)
